# initial kernel scaffold (unmeasured)
import jax
import jax.numpy as jnp
from jax import lax
from jax.experimental import pallas as pl
from jax.experimental.pallas import tpu as pltpu

N_DEV = 8
SQ = 2048
SKV = 2048
D_MODEL = 1024
H_PER = 8
DH = 128
DQ = H_PER * DH
SCALE = 0.08838834764831843
CHUNK = SQ // N_DEV
QBLK = 256


def kernel(x, Wq, K_ext, V_ext, Wo):
    my = lax.axis_index("i")
    xb = x.reshape(SQ, D_MODEL).astype(jnp.bfloat16)
    wq = lax.dynamic_slice(Wq, (0, my * DQ), (D_MODEL, DQ)).astype(jnp.bfloat16)
    wo = lax.dynamic_slice(Wo, (my * DQ, 0), (DQ, D_MODEL)).astype(jnp.bfloat16)
    k = K_ext.reshape(SKV, DQ).astype(jnp.bfloat16)
    v = V_ext.reshape(SKV, DQ).astype(jnp.bfloat16)

    def body(x_ref, wq_ref, k_ref, v_ref, wo_ref, out_ref,
             q_buf, ctx_buf, rs_buf, rs_send, rs_recv, ag_send, ag_recv):
        me = lax.axis_index("i")
        right = lax.rem(me + 1, N_DEV)

        q = jnp.dot(x_ref[...], wq_ref[...], preferred_element_type=jnp.float32)
        q_buf[...] = (q * SCALE).astype(jnp.bfloat16)

        def qblock(c, carry):
            rows = pl.ds(c * QBLK, QBLK)
            for h in range(H_PER):
                cols = slice(h * DH, (h + 1) * DH)
                s = lax.dot_general(
                    q_buf[rows, cols], k_ref[:, cols],
                    (((1,), (1,)), ((), ())),
                    preferred_element_type=jnp.float32)
                qi = c * QBLK + lax.broadcasted_iota(jnp.int32, (QBLK, SKV), 0)
                ki = lax.broadcasted_iota(jnp.int32, (QBLK, SKV), 1)
                qb = qi // 64
                kb = ki // 64
                mask = (qb == kb) | (kb == 0) | (lax.rem(qb + kb, 3) == 0)
                s = jnp.where(mask, s, -1e9)
                m = jnp.max(s, axis=1, keepdims=True)
                w = jnp.exp(s - m)
                ws = jnp.sum(w, axis=1, keepdims=True)
                wn = (w / ws).astype(jnp.bfloat16)
                ctx_buf[rows, cols] = jnp.dot(
                    wn, v_ref[:, cols], preferred_element_type=jnp.float32
                ).astype(jnp.bfloat16)
            return carry

        lax.fori_loop(0, SQ // QBLK, qblock, 0)

        out_ref[...] = jnp.dot(ctx_buf[...], wo_ref[...],
                               preferred_element_type=jnp.float32)

        for step in range(N_DEV - 1):
            send_c = lax.rem(me + N_DEV - step, N_DEV)
            recv_c = lax.rem(me + N_DEV - step - 1, N_DEV)
            rdma = pltpu.make_async_remote_copy(
                src_ref=out_ref.at[pl.ds(send_c * CHUNK, CHUNK), :],
                dst_ref=rs_buf.at[step],
                send_sem=rs_send.at[step],
                recv_sem=rs_recv.at[step],
                device_id=(right,),
                device_id_type=pl.DeviceIdType.MESH,
            )
            rdma.start()
            rdma.wait()
            roff = recv_c * CHUNK
            out_ref[pl.ds(roff, CHUNK), :] = (
                out_ref[pl.ds(roff, CHUNK), :] + rs_buf[step]
            )

        for step in range(N_DEV - 1):
            send_c = lax.rem(me + 1 + N_DEV - step, N_DEV)
            rdma = pltpu.make_async_remote_copy(
                src_ref=out_ref.at[pl.ds(send_c * CHUNK, CHUNK), :],
                dst_ref=out_ref.at[pl.ds(send_c * CHUNK, CHUNK), :],
                send_sem=ag_send.at[step],
                recv_sem=ag_recv.at[step],
                device_id=(right,),
                device_id_type=pl.DeviceIdType.MESH,
            )
            rdma.start()
            rdma.wait()

    out = pl.pallas_call(
        body,
        out_shape=jax.ShapeDtypeStruct((SQ, D_MODEL), jnp.float32),
        in_specs=[pl.BlockSpec(memory_space=pltpu.VMEM)] * 5,
        out_specs=pl.BlockSpec(memory_space=pltpu.VMEM),
        scratch_shapes=[
            pltpu.VMEM((SQ, DQ), jnp.bfloat16),
            pltpu.VMEM((SQ, DQ), jnp.bfloat16),
            pltpu.VMEM((N_DEV - 1, CHUNK, D_MODEL), jnp.float32),
            pltpu.SemaphoreType.DMA((N_DEV - 1,)),
            pltpu.SemaphoreType.DMA((N_DEV - 1,)),
            pltpu.SemaphoreType.DMA((N_DEV - 1,)),
            pltpu.SemaphoreType.DMA((N_DEV - 1,)),
        ],
    )(xb, wq, k, v, wo)
    return out.reshape(1, SQ, D_MODEL)


# baseline (device time: 325192 ns/iter reference)
import jax
import jax.numpy as jnp
from jax import lax
from jax.experimental import pallas as pl
from jax.experimental.pallas import tpu as pltpu

N_DEV = 8
SQ = 2048
SKV = 2048
D_MODEL = 1024
H_PER = 8
DH = 128
DQ = H_PER * DH
SCALE = 0.08838834764831843
CHUNK = SQ // N_DEV
QBLK = 256


def kernel(x, Wq, K_ext, V_ext, Wo):
    my = lax.axis_index("i")
    xb = x.reshape(SQ, D_MODEL).astype(jnp.bfloat16)
    wq = lax.dynamic_slice(Wq, (0, my * DQ), (D_MODEL, DQ)).astype(jnp.bfloat16)
    wo = lax.dynamic_slice(Wo, (my * DQ, 0), (DQ, D_MODEL)).astype(jnp.bfloat16)
    k = K_ext.reshape(SKV, DQ).astype(jnp.bfloat16)
    v = V_ext.reshape(SKV, DQ).astype(jnp.bfloat16)

    def body(x_ref, wq_ref, k_ref, v_ref, wo_ref, out_ref,
             q_buf, ctx_buf, rs_buf, rs_send, rs_recv, ag_send, ag_recv):
        me = lax.axis_index("i")
        right = lax.rem(me + 1, N_DEV)

        def qblock(c, carry):
            rows = pl.ds(c * QBLK, QBLK)
            q_buf[rows, :] = (
                jnp.dot(x_ref[rows, :], wq_ref[...],
                        preferred_element_type=jnp.float32) * SCALE
            ).astype(jnp.bfloat16)
            qi = c * QBLK + lax.broadcasted_iota(jnp.int32, (QBLK, SKV), 0)
            ki = lax.broadcasted_iota(jnp.int32, (QBLK, SKV), 1)
            qb = qi // 64
            kb = ki // 64
            mask = (qb == kb) | (kb == 0) | (lax.rem(qb + kb, 3) == 0)
            for h in range(H_PER):
                cols = slice(h * DH, (h + 1) * DH)
                s = lax.dot_general(
                    q_buf[rows, cols], k_ref[:, cols],
                    (((1,), (1,)), ((), ())),
                    preferred_element_type=jnp.float32)
                s = jnp.where(mask, s, -1e9)
                m = jnp.max(s, axis=1, keepdims=True)
                w = jnp.exp(s - m)
                ws = jnp.sum(w, axis=1, keepdims=True)
                wn = (w / ws).astype(jnp.bfloat16)
                ctx_buf[rows, cols] = jnp.dot(
                    wn, v_ref[:, cols], preferred_element_type=jnp.float32
                ).astype(jnp.bfloat16)
            return carry

        lax.fori_loop(0, SQ // QBLK, qblock, 0)

        out_ref[...] = jnp.dot(ctx_buf[...], wo_ref[...],
                               preferred_element_type=jnp.float32)

        for step in range(N_DEV - 1):
            send_c = lax.rem(me + N_DEV - step, N_DEV)
            recv_c = lax.rem(me + N_DEV - step - 1, N_DEV)
            rdma = pltpu.make_async_remote_copy(
                src_ref=out_ref.at[pl.ds(send_c * CHUNK, CHUNK), :],
                dst_ref=rs_buf.at[step],
                send_sem=rs_send.at[step],
                recv_sem=rs_recv.at[step],
                device_id=(right,),
                device_id_type=pl.DeviceIdType.MESH,
            )
            rdma.start()
            rdma.wait()
            roff = recv_c * CHUNK
            out_ref[pl.ds(roff, CHUNK), :] = (
                out_ref[pl.ds(roff, CHUNK), :] + rs_buf[step]
            )

        for step in range(N_DEV - 1):
            send_c = lax.rem(me + 1 + N_DEV - step, N_DEV)
            rdma = pltpu.make_async_remote_copy(
                src_ref=out_ref.at[pl.ds(send_c * CHUNK, CHUNK), :],
                dst_ref=out_ref.at[pl.ds(send_c * CHUNK, CHUNK), :],
                send_sem=ag_send.at[step],
                recv_sem=ag_recv.at[step],
                device_id=(right,),
                device_id_type=pl.DeviceIdType.MESH,
            )
            rdma.start()
            rdma.wait()

    out = pl.pallas_call(
        body,
        out_shape=jax.ShapeDtypeStruct((SQ, D_MODEL), jnp.float32),
        in_specs=[pl.BlockSpec(memory_space=pltpu.VMEM)] * 5,
        out_specs=pl.BlockSpec(memory_space=pltpu.VMEM),
        scratch_shapes=[
            pltpu.VMEM((SQ, DQ), jnp.bfloat16),
            pltpu.VMEM((SQ, DQ), jnp.bfloat16),
            pltpu.VMEM((N_DEV - 1, CHUNK, D_MODEL), jnp.float32),
            pltpu.SemaphoreType.DMA((N_DEV - 1,)),
            pltpu.SemaphoreType.DMA((N_DEV - 1,)),
            pltpu.SemaphoreType.DMA((N_DEV - 1,)),
            pltpu.SemaphoreType.DMA((N_DEV - 1,)),
        ],
        compiler_params=pltpu.CompilerParams(
            vmem_limit_bytes=100 * 1024 * 1024,
        ),
    )(xb, wq, k, v, wo)
    return out.reshape(1, SQ, D_MODEL)


# device time: 247179 ns/iter; 1.3156x vs baseline; 1.3156x over previous
import jax
import jax.numpy as jnp
from jax import lax
from jax.experimental import pallas as pl
from jax.experimental.pallas import tpu as pltpu

N_DEV = 8
SQ = 2048
SKV = 2048
D_MODEL = 1024
H_PER = 8
DH = 128
DQ = H_PER * DH
SCALE = 0.08838834764831843
CHUNK = SQ // N_DEV
QBLK = 256


def kernel(x, Wq, K_ext, V_ext, Wo):
    my = lax.axis_index("i")
    xb = x.reshape(SQ, D_MODEL).astype(jnp.bfloat16)
    wq = lax.dynamic_slice(Wq, (0, my * DQ), (D_MODEL, DQ)).astype(jnp.bfloat16)
    wo = lax.dynamic_slice(Wo, (my * DQ, 0), (DQ, D_MODEL)).astype(jnp.bfloat16)
    k = K_ext.reshape(SKV, DQ).astype(jnp.bfloat16)
    v = V_ext.reshape(SKV, DQ).astype(jnp.bfloat16)

    def body(x_ref, wq_ref, k_ref, v_ref, wo_ref, out_ref,
             q_buf, ctx_buf, snd_buf, rs_buf, ag_buf,
             rs_send, rs_recv, ag_send, ag_recv):
        me = lax.axis_index("i")
        right = lax.rem(me + 1, N_DEV)

        def qblock(c, carry):
            rows = pl.ds(c * QBLK, QBLK)
            q_buf[rows, :] = (
                jnp.dot(x_ref[rows, :], wq_ref[...],
                        preferred_element_type=jnp.float32) * SCALE
            ).astype(jnp.bfloat16)
            qi = c * QBLK + lax.broadcasted_iota(jnp.int32, (QBLK, SKV), 0)
            ki = lax.broadcasted_iota(jnp.int32, (QBLK, SKV), 1)
            qb = qi // 64
            kb = ki // 64
            mask = (qb == kb) | (kb == 0) | (lax.rem(qb + kb, 3) == 0)
            for h in range(H_PER):
                cols = slice(h * DH, (h + 1) * DH)
                s = lax.dot_general(
                    q_buf[rows, cols], k_ref[:, cols],
                    (((1,), (1,)), ((), ())),
                    preferred_element_type=jnp.float32)
                s = jnp.where(mask, s, -1e9)
                m = jnp.max(s, axis=1, keepdims=True)
                w = jnp.exp(s - m)
                ws = jnp.sum(w, axis=1, keepdims=True)
                wn = (w / ws).astype(jnp.bfloat16)
                ctx_buf[rows, cols] = jnp.dot(
                    wn, v_ref[:, cols], preferred_element_type=jnp.float32
                ).astype(jnp.bfloat16)
            return carry

        lax.fori_loop(0, SQ // QBLK, qblock, 0)

        out_ref[...] = jnp.dot(ctx_buf[...], wo_ref[...],
                               preferred_element_type=jnp.float32)

        for step in range(N_DEV - 1):
            send_c = lax.rem(me + N_DEV - step, N_DEV)
            recv_c = lax.rem(me + N_DEV - step - 1, N_DEV)
            slot = step % 2
            snd_buf[slot] = out_ref[pl.ds(send_c * CHUNK, CHUNK), :].astype(
                jnp.bfloat16)
            rdma = pltpu.make_async_remote_copy(
                src_ref=snd_buf.at[slot],
                dst_ref=rs_buf.at[step],
                send_sem=rs_send.at[step],
                recv_sem=rs_recv.at[step],
                device_id=(right,),
                device_id_type=pl.DeviceIdType.MESH,
            )
            rdma.start()
            rdma.wait()
            roff = recv_c * CHUNK
            out_ref[pl.ds(roff, CHUNK), :] = (
                out_ref[pl.ds(roff, CHUNK), :]
                + rs_buf[step].astype(jnp.float32)
            )

        own = lax.rem(me + 1, N_DEV)
        ag_buf[pl.ds(own * CHUNK, CHUNK), :] = out_ref[
            pl.ds(own * CHUNK, CHUNK), :].astype(jnp.bfloat16)
        for step in range(N_DEV - 1):
            send_c = lax.rem(me + 1 + N_DEV - step, N_DEV)
            recv_c = lax.rem(me + N_DEV - step, N_DEV)
            rdma = pltpu.make_async_remote_copy(
                src_ref=ag_buf.at[pl.ds(send_c * CHUNK, CHUNK), :],
                dst_ref=ag_buf.at[pl.ds(send_c * CHUNK, CHUNK), :],
                send_sem=ag_send.at[step],
                recv_sem=ag_recv.at[step],
                device_id=(right,),
                device_id_type=pl.DeviceIdType.MESH,
            )
            rdma.start()
            rdma.wait()
            roff = recv_c * CHUNK
            out_ref[pl.ds(roff, CHUNK), :] = ag_buf[
                pl.ds(roff, CHUNK), :].astype(jnp.float32)

    out = pl.pallas_call(
        body,
        out_shape=jax.ShapeDtypeStruct((SQ, D_MODEL), jnp.float32),
        in_specs=[pl.BlockSpec(memory_space=pltpu.VMEM)] * 5,
        out_specs=pl.BlockSpec(memory_space=pltpu.VMEM),
        scratch_shapes=[
            pltpu.VMEM((SQ, DQ), jnp.bfloat16),
            pltpu.VMEM((SQ, DQ), jnp.bfloat16),
            pltpu.VMEM((2, CHUNK, D_MODEL), jnp.bfloat16),
            pltpu.VMEM((N_DEV - 1, CHUNK, D_MODEL), jnp.bfloat16),
            pltpu.VMEM((SQ, D_MODEL), jnp.bfloat16),
            pltpu.SemaphoreType.DMA((N_DEV - 1,)),
            pltpu.SemaphoreType.DMA((N_DEV - 1,)),
            pltpu.SemaphoreType.DMA((N_DEV - 1,)),
            pltpu.SemaphoreType.DMA((N_DEV - 1,)),
        ],
        compiler_params=pltpu.CompilerParams(
            vmem_limit_bytes=100 * 1024 * 1024,
        ),
    )(xb, wq, k, v, wo)
    return out.reshape(1, SQ, D_MODEL)


# device time: 195114 ns/iter; 1.6667x vs baseline; 1.2668x over previous
import jax
import jax.numpy as jnp
from jax import lax
from jax.experimental import pallas as pl
from jax.experimental.pallas import tpu as pltpu

N_DEV = 8
SQ = 2048
SKV = 2048
D_MODEL = 1024
H_PER = 8
DH = 128
DQ = H_PER * DH
SCALE = 0.08838834764831843
CHUNK = SQ // N_DEV
QBLK = 256


def kernel(x, Wq, K_ext, V_ext, Wo):
    my = lax.axis_index("i")
    xb = x.reshape(SQ, D_MODEL).astype(jnp.bfloat16)
    wq = lax.dynamic_slice(Wq, (0, my * DQ), (D_MODEL, DQ)).astype(jnp.bfloat16)
    wo = lax.dynamic_slice(Wo, (my * DQ, 0), (DQ, D_MODEL)).astype(jnp.bfloat16)
    k = K_ext.reshape(SKV, DQ).astype(jnp.bfloat16)
    v = V_ext.reshape(SKV, DQ).astype(jnp.bfloat16)

    def body(x_ref, wq_ref, k_ref, v_ref, wo_ref, out_ref,
             q_buf, ctx_buf, snd_buf, rs_buf, ag_buf,
             rs_send, rs_recv, ag_send, ag_recv):
        me = lax.axis_index("i")
        right = lax.rem(me + 1, N_DEV)

        def rs_step(s, carry):
            c = lax.rem(me + N_DEV - s, N_DEV)
            rows = pl.ds(c * QBLK, QBLK)
            q_buf[rows, :] = (
                jnp.dot(x_ref[rows, :], wq_ref[...],
                        preferred_element_type=jnp.float32) * SCALE
            ).astype(jnp.bfloat16)
            qi = c * QBLK + lax.broadcasted_iota(jnp.int32, (QBLK, SKV), 0)
            ki = lax.broadcasted_iota(jnp.int32, (QBLK, SKV), 1)
            qb = qi // 64
            kb = ki // 64
            mask = (qb == kb) | (kb == 0) | (lax.rem(qb + kb, 3) == 0)
            for h in range(H_PER):
                cols = slice(h * DH, (h + 1) * DH)
                sc = lax.dot_general(
                    q_buf[rows, cols], k_ref[:, cols],
                    (((1,), (1,)), ((), ())),
                    preferred_element_type=jnp.float32)
                sc = jnp.where(mask, sc, -1e9)
                m = jnp.max(sc, axis=1, keepdims=True)
                w = jnp.exp(sc - m)
                ws = jnp.sum(w, axis=1, keepdims=True)
                wn = (w / ws).astype(jnp.bfloat16)
                ctx_buf[rows, cols] = jnp.dot(
                    wn, v_ref[:, cols], preferred_element_type=jnp.float32
                ).astype(jnp.bfloat16)
            out_ref[rows, :] = jnp.dot(ctx_buf[rows, :], wo_ref[...],
                                       preferred_element_type=jnp.float32)

            sm1 = lax.max(s - 1, 0)

            @pl.when(s > 0)
            def _():
                prev = pltpu.make_async_remote_copy(
                    src_ref=snd_buf.at[lax.rem(sm1, 2)],
                    dst_ref=rs_buf.at[sm1],
                    send_sem=rs_send.at[sm1],
                    recv_sem=rs_recv.at[sm1],
                    device_id=(right,),
                    device_id_type=pl.DeviceIdType.MESH,
                )
                prev.wait()
                out_ref[rows, :] = (
                    out_ref[rows, :] + rs_buf[sm1].astype(jnp.float32)
                )

            @pl.when(s < N_DEV - 1)
            def _():
                slot = lax.rem(s, 2)
                snd_buf[slot] = out_ref[rows, :].astype(jnp.bfloat16)
                rdma = pltpu.make_async_remote_copy(
                    src_ref=snd_buf.at[slot],
                    dst_ref=rs_buf.at[s],
                    send_sem=rs_send.at[s],
                    recv_sem=rs_recv.at[s],
                    device_id=(right,),
                    device_id_type=pl.DeviceIdType.MESH,
                )
                rdma.start()

            return carry

        lax.fori_loop(0, N_DEV, rs_step, 0)

        own = lax.rem(me + 1, N_DEV)
        ag_buf[pl.ds(own * CHUNK, CHUNK), :] = out_ref[
            pl.ds(own * CHUNK, CHUNK), :].astype(jnp.bfloat16)
        for step in range(N_DEV - 1):
            send_c = lax.rem(me + 1 + N_DEV - step, N_DEV)
            recv_c = lax.rem(me + N_DEV - step, N_DEV)
            rdma = pltpu.make_async_remote_copy(
                src_ref=ag_buf.at[pl.ds(send_c * CHUNK, CHUNK), :],
                dst_ref=ag_buf.at[pl.ds(send_c * CHUNK, CHUNK), :],
                send_sem=ag_send.at[step],
                recv_sem=ag_recv.at[step],
                device_id=(right,),
                device_id_type=pl.DeviceIdType.MESH,
            )
            rdma.start()
            rdma.wait()
            roff = recv_c * CHUNK
            out_ref[pl.ds(roff, CHUNK), :] = ag_buf[
                pl.ds(roff, CHUNK), :].astype(jnp.float32)

    out = pl.pallas_call(
        body,
        out_shape=jax.ShapeDtypeStruct((SQ, D_MODEL), jnp.float32),
        in_specs=[pl.BlockSpec(memory_space=pltpu.VMEM)] * 5,
        out_specs=pl.BlockSpec(memory_space=pltpu.VMEM),
        scratch_shapes=[
            pltpu.VMEM((SQ, DQ), jnp.bfloat16),
            pltpu.VMEM((SQ, DQ), jnp.bfloat16),
            pltpu.VMEM((2, CHUNK, D_MODEL), jnp.bfloat16),
            pltpu.VMEM((N_DEV - 1, CHUNK, D_MODEL), jnp.bfloat16),
            pltpu.VMEM((SQ, D_MODEL), jnp.bfloat16),
            pltpu.SemaphoreType.DMA((N_DEV - 1,)),
            pltpu.SemaphoreType.DMA((N_DEV - 1,)),
            pltpu.SemaphoreType.DMA((N_DEV - 1,)),
            pltpu.SemaphoreType.DMA((N_DEV - 1,)),
        ],
        compiler_params=pltpu.CompilerParams(
            vmem_limit_bytes=100 * 1024 * 1024,
        ),
    )(xb, wq, k, v, wo)
    return out.reshape(1, SQ, D_MODEL)


# device time: 142569 ns/iter; 2.2809x vs baseline; 1.3686x over previous
import jax
import jax.numpy as jnp
from jax import lax
from jax.experimental import pallas as pl
from jax.experimental.pallas import tpu as pltpu

N_DEV = 8
SQ = 2048
SKV = 2048
D_MODEL = 1024
H_PER = 8
DH = 128
DQ = H_PER * DH
SCALE = 0.08838834764831843
CHUNK = SQ // N_DEV
QBLK = CHUNK


def kernel(x, Wq, K_ext, V_ext, Wo):
    my = lax.axis_index("i")
    xb = x.reshape(SQ, D_MODEL).astype(jnp.bfloat16)
    wq = lax.dynamic_slice(Wq, (0, my * DQ), (D_MODEL, DQ)).astype(jnp.bfloat16)
    wo = lax.dynamic_slice(Wo, (my * DQ, 0), (DQ, D_MODEL)).astype(jnp.bfloat16)
    k = K_ext.reshape(SKV, DQ).astype(jnp.bfloat16)
    v = V_ext.reshape(SKV, DQ).astype(jnp.bfloat16)

    def body(x_ref, wq_ref, k_ref, v_ref, wo_ref, out_ref,
             q_buf, ctx_buf, rs_buf, rs_send, rs_recv,
             agr_send, agr_recv, agl_send, agl_recv):
        me = lax.axis_index("i")
        right = lax.rem(me + 1, N_DEV)
        left = lax.rem(me + N_DEV - 1, N_DEV)

        def rs_step(s, carry):
            c = lax.rem(me + N_DEV - s, N_DEV)
            rows = pl.ds(c * QBLK, QBLK)
            q_buf[rows, :] = (
                jnp.dot(x_ref[rows, :], wq_ref[...],
                        preferred_element_type=jnp.float32) * SCALE
            ).astype(jnp.bfloat16)
            qi = c * QBLK + lax.broadcasted_iota(jnp.int32, (QBLK, SKV), 0)
            ki = lax.broadcasted_iota(jnp.int32, (QBLK, SKV), 1)
            qb = qi // 64
            kb = ki // 64
            mask = (qb == kb) | (kb == 0) | (lax.rem(qb + kb, 3) == 0)
            for h in range(H_PER):
                cols = slice(h * DH, (h + 1) * DH)
                sc = lax.dot_general(
                    q_buf[rows, cols], k_ref[:, cols],
                    (((1,), (1,)), ((), ())),
                    preferred_element_type=jnp.float32)
                w = jnp.exp(jnp.where(mask, sc, -1e9))
                ws = jnp.sum(w, axis=1, keepdims=True)
                ctx = jnp.dot(w.astype(jnp.bfloat16), v_ref[:, cols],
                              preferred_element_type=jnp.float32)
                ctx_buf[rows, cols] = (ctx / ws).astype(jnp.bfloat16)
            out_ref[rows, :] = jnp.dot(
                ctx_buf[rows, :], wo_ref[...],
                preferred_element_type=jnp.float32).astype(jnp.bfloat16)

            sm1 = lax.max(s - 1, 0)
            cm1 = lax.rem(me + N_DEV - sm1, N_DEV)

            @pl.when(s > 0)
            def _():
                prev = pltpu.make_async_remote_copy(
                    src_ref=out_ref.at[pl.ds(cm1 * QBLK, QBLK), :],
                    dst_ref=rs_buf.at[sm1],
                    send_sem=rs_send.at[sm1],
                    recv_sem=rs_recv.at[sm1],
                    device_id=(right,),
                    device_id_type=pl.DeviceIdType.MESH,
                )
                prev.wait()
                out_ref[rows, :] = out_ref[rows, :] + rs_buf[sm1]

            @pl.when(s < N_DEV - 1)
            def _():
                rdma = pltpu.make_async_remote_copy(
                    src_ref=out_ref.at[rows, :],
                    dst_ref=rs_buf.at[s],
                    send_sem=rs_send.at[s],
                    recv_sem=rs_recv.at[s],
                    device_id=(right,),
                    device_id_type=pl.DeviceIdType.MESH,
                )
                rdma.start()

            return carry

        lax.fori_loop(0, N_DEV, rs_step, 0)

        for t in range(4):
            sc_r = lax.rem(me + 1 + N_DEV - t, N_DEV)
            rows_r = pl.ds(sc_r * CHUNK, CHUNK)
            rd_r = pltpu.make_async_remote_copy(
                src_ref=out_ref.at[rows_r, :],
                dst_ref=out_ref.at[rows_r, :],
                send_sem=agr_send.at[t],
                recv_sem=agr_recv.at[t],
                device_id=(right,),
                device_id_type=pl.DeviceIdType.MESH,
            )
            rd_r.start()
            if t < 3:
                sc_l = lax.rem(me + 1 + t, N_DEV)
                rows_l = pl.ds(sc_l * CHUNK, CHUNK)
                rd_l = pltpu.make_async_remote_copy(
                    src_ref=out_ref.at[rows_l, :],
                    dst_ref=out_ref.at[rows_l, :],
                    send_sem=agl_send.at[t],
                    recv_sem=agl_recv.at[t],
                    device_id=(left,),
                    device_id_type=pl.DeviceIdType.MESH,
                )
                rd_l.start()
            rd_r.wait()
            if t < 3:
                rd_l.wait()

    out = pl.pallas_call(
        body,
        out_shape=jax.ShapeDtypeStruct((SQ, D_MODEL), jnp.bfloat16),
        in_specs=[pl.BlockSpec(memory_space=pltpu.VMEM)] * 5,
        out_specs=pl.BlockSpec(memory_space=pltpu.VMEM),
        scratch_shapes=[
            pltpu.VMEM((SQ, DQ), jnp.bfloat16),
            pltpu.VMEM((SQ, DQ), jnp.bfloat16),
            pltpu.VMEM((N_DEV - 1, CHUNK, D_MODEL), jnp.bfloat16),
            pltpu.SemaphoreType.DMA((N_DEV - 1,)),
            pltpu.SemaphoreType.DMA((N_DEV - 1,)),
            pltpu.SemaphoreType.DMA((4,)),
            pltpu.SemaphoreType.DMA((4,)),
            pltpu.SemaphoreType.DMA((4,)),
            pltpu.SemaphoreType.DMA((4,)),
        ],
        compiler_params=pltpu.CompilerParams(
            vmem_limit_bytes=100 * 1024 * 1024,
        ),
    )(xb, wq, k, v, wo)
    return out.reshape(1, SQ, D_MODEL)
